# baseline (speedup 1.0000x reference)
import functools

import jax
import jax.numpy as jnp
from jax import lax
from jax.experimental import pallas as pl
from jax.experimental.pallas import tpu as pltpu
from jax.experimental.pallas import tpu_sc as plsc

B, N = 4096, 64
TOTAL = B * N * 2
NC, NS = 2, 16
NW = NC * NS
PER_W = TOTAL // NW
DMA_B = 128
N_DMA = PER_W // DMA_B
FIRE = 8


HALF = PER_W // 2


def _sc_gather_body(table_hbm, idx_hbm, out_hbm, idx_v, rows_v, sem):
    wid = lax.axis_index("s") * NC + lax.axis_index("c")
    pltpu.sync_copy(idx_hbm.at[wid], idx_v)

    for half in range(2):
        base_dma = half * (N_DMA // 2)

        def body(j, carry):
            cps = [
                pltpu.async_copy(
                    table_hbm.at[idx_v.at[base_dma + j * FIRE + t]],
                    rows_v.at[pl.ds((j * FIRE + t) * DMA_B, DMA_B), :],
                    sem,
                )
                for t in range(FIRE)
            ]
            for cp in cps:
                cp.wait()
            return carry

        lax.fori_loop(0, N_DMA // 2 // FIRE, body, 0)
        pltpu.sync_copy(
            rows_v, out_hbm.at[pl.ds(wid * PER_W + half * HALF, HALF), :]
        )


@functools.cache
def _sc_gather():
    return pl.kernel(
        _sc_gather_body,
        out_type=jax.ShapeDtypeStruct((TOTAL, 4), jnp.float32),
        mesh=plsc.VectorSubcoreMesh(
            core_axis_name="c", subcore_axis_name="s",
            num_cores=NC, num_subcores=NS,
        ),
        scratch_types=[
            pltpu.VMEM((N_DMA, DMA_B), jnp.int32),
            pltpu.VMEM((HALF, 4), jnp.float32),
            pltpu.SemaphoreType.DMA,
        ],
        compiler_params=pltpu.CompilerParams(use_tc_tiling_on_sc=False),
    )


BB = 512


def _sigmoid(z):
    return 1.0 / (1.0 + jnp.exp(-z))


def _tc_body(planes_ref, d_ref, w0_ref, w1_ref, sigma_ref, color_ref):
    p = planes_ref[...]
    w0 = w0_ref[...]
    w1 = w1_ref[...]

    sigma = _sigmoid(p[0] * p[4])
    sigma_ref[...] = sigma

    dproj = lax.dot_general(
        d_ref[...], w0[:, :16], (((1,), (1,)), ((), ())),
        preferred_element_type=jnp.float32,
    )

    hs = []
    for j in range(8):
        acc = dproj[:, j][:, None]
        for k in range(6):
            plane = 1 + k if k < 3 else 2 + k
            acc = acc + p[plane] * w0[j, 16 + k]
        hs.append(jnp.maximum(acc, 0.0))

    cs = []
    for i in range(3):
        accc = hs[0] * w1[i, 0]
        for j in range(1, 8):
            accc = accc + hs[j] * w1[i, j]
        cs.append(_sigmoid(accc))

    cp = 1.0 - sigma
    for sh in (1, 2, 4, 8, 16, 32):
        shifted = jnp.concatenate(
            [jnp.ones((BB, sh), jnp.float32), cp[:, :-sh]], axis=1
        )
        cp = cp * shifted
    tb = jnp.concatenate([jnp.ones((BB, 1), jnp.float32), cp[:, :-1]], axis=1)

    cw = tb * sigma
    color_ref[...] = jnp.stack(
        [jnp.sum(cw * cs[i], axis=1) for i in range(3)], axis=1
    )


def _tc_dense(planes, d, Wc0, Wc1):
    return pl.pallas_call(
        _tc_body,
        grid=(B // BB,),
        in_specs=[
            pl.BlockSpec((8, BB, N), lambda i: (0, i, 0)),
            pl.BlockSpec((BB, 16), lambda i: (i, 0)),
            pl.BlockSpec((8, 22), lambda i: (0, 0)),
            pl.BlockSpec((3, 8), lambda i: (0, 0)),
        ],
        out_specs=[
            pl.BlockSpec((BB, N), lambda i: (i, 0)),
            pl.BlockSpec((BB, 3), lambda i: (i, 0)),
        ],
        out_shape=[
            jax.ShapeDtypeStruct((B, N), jnp.float32),
            jax.ShapeDtypeStruct((B, 3), jnp.float32),
        ],
    )(planes, d, Wc0, Wc1)


def kernel(x, d, gridWeight, Wc0, Wc1):
    idx = x.reshape(NW, N_DMA, DMA_B)
    feat = _sc_gather()(gridWeight, idx)
    planes = jnp.transpose(feat.reshape(B, N, 8), (2, 0, 1))
    sigma, color = _tc_dense(planes, d, Wc0, Wc1)
    return (sigma, color)


# speedup vs baseline: 1.2009x; 1.2009x over previous
import functools

import jax
import jax.numpy as jnp
from jax import lax
from jax.experimental import pallas as pl
from jax.experimental.pallas import tpu as pltpu
from jax.experimental.pallas import tpu_sc as plsc

B, N = 4096, 64
VOCAB = 1000000
TOTAL = B * N * 2
SAMPLES = B * N
NC, NS = 2, 16
NW = NC * NS
PER_W = TOTAL // NW
SAMP_W = SAMPLES // NW
DMA_B = 128
N_DMA = PER_W // DMA_B
FIRE = 8

CH_S = 2048
CH_R = CH_S * 2
N_CHUNK = PER_W // CH_R
DMA_PER_CHUNK = CH_R // DMA_B


def _sc_gather_body(table_hbm, idx_hbm, out_hbm, idx_v, rows_v, plane_v, sem):
    wid = lax.axis_index("s") * NC + lax.axis_index("c")
    pltpu.sync_copy(idx_hbm.at[wid], idx_v)
    lane = lax.iota(jnp.int32, 16)

    for chunk in range(N_CHUNK):
        dma0 = chunk * DMA_PER_CHUNK

        def gbody(g, carry):
            cps = [
                pltpu.async_copy(
                    table_hbm.at[idx_v.at[dma0 + g * FIRE + t]],
                    rows_v.at[pl.ds((g * FIRE + t) * DMA_B, DMA_B), :],
                    sem,
                )
                for t in range(FIRE)
            ]
            for cp in cps:
                cp.wait()
            return carry

        lax.fori_loop(0, DMA_PER_CHUNK // FIRE, gbody, 0)

        def rbody(j, carry):
            row0 = j * 32 + 2 * lane
            row1 = row0 + 1
            c0 = jnp.zeros((16,), jnp.int32)
            s1 = plsc.load_gather(rows_v, [row0, c0])
            s2 = plsc.load_gather(rows_v, [row1, c0])
            plane_v[0, pl.ds(j * 16, 16)] = s1 * s2
            for k in range(3):
                ck = jnp.full((16,), k + 1, jnp.int32)
                plane_v[1 + k, pl.ds(j * 16, 16)] = plsc.load_gather(
                    rows_v, [row0, ck]
                )
                plane_v[4 + k, pl.ds(j * 16, 16)] = plsc.load_gather(
                    rows_v, [row1, ck]
                )
            return carry

        lax.fori_loop(0, CH_S // 16, rbody, 0)

        base = wid * SAMP_W + chunk * CH_S
        cps = [
            pltpu.async_copy(plane_v.at[k], out_hbm.at[k, pl.ds(base, CH_S)], sem)
            for k in range(7)
        ]
        for cp in cps:
            cp.wait()


@functools.cache
def _sc_gather():
    return pl.kernel(
        _sc_gather_body,
        out_type=jax.ShapeDtypeStruct((7, SAMPLES), jnp.float32),
        mesh=plsc.VectorSubcoreMesh(
            core_axis_name="c", subcore_axis_name="s",
            num_cores=NC, num_subcores=NS,
        ),
        scratch_types=[
            pltpu.VMEM((N_DMA, DMA_B), jnp.int32),
            pltpu.VMEM((CH_R, 8), jnp.float32),
            pltpu.VMEM((7, CH_S), jnp.float32),
            pltpu.SemaphoreType.DMA,
        ],
        compiler_params=pltpu.CompilerParams(
            use_tc_tiling_on_sc=False, needs_layout_passes=False
        ),
    )


BB = 512


def _sigmoid(z):
    return 1.0 / (1.0 + jnp.exp(-z))


def _tc_body(planes_ref, d_ref, w0_ref, w1_ref, sigma_ref, color_ref):
    p = planes_ref[...]
    w0 = w0_ref[...]
    w1 = w1_ref[...]

    sigma = _sigmoid(p[0])
    sigma_ref[...] = sigma

    dproj = lax.dot_general(
        d_ref[...], w0[:, :16], (((1,), (1,)), ((), ())),
        preferred_element_type=jnp.float32,
    )

    hs = []
    for j in range(8):
        acc = dproj[:, j][:, None]
        for k in range(6):
            acc = acc + p[1 + k] * w0[j, 16 + k]
        hs.append(jnp.maximum(acc, 0.0))

    cs = []
    for i in range(3):
        accc = hs[0] * w1[i, 0]
        for j in range(1, 8):
            accc = accc + hs[j] * w1[i, j]
        cs.append(_sigmoid(accc))

    cp = 1.0 - sigma
    for sh in (1, 2, 4, 8, 16, 32):
        shifted = jnp.concatenate(
            [jnp.ones((BB, sh), jnp.float32), cp[:, :-sh]], axis=1
        )
        cp = cp * shifted
    tb = jnp.concatenate([jnp.ones((BB, 1), jnp.float32), cp[:, :-1]], axis=1)

    cw = tb * sigma
    color_ref[...] = jnp.stack(
        [jnp.sum(cw * cs[i], axis=1) for i in range(3)], axis=1
    )


def _tc_dense(planes, d, Wc0, Wc1):
    return pl.pallas_call(
        _tc_body,
        grid=(B // BB,),
        in_specs=[
            pl.BlockSpec((7, BB, N), lambda i: (0, i, 0)),
            pl.BlockSpec((BB, 16), lambda i: (i, 0)),
            pl.BlockSpec((8, 22), lambda i: (0, 0)),
            pl.BlockSpec((3, 8), lambda i: (0, 0)),
        ],
        out_specs=[
            pl.BlockSpec((BB, N), lambda i: (i, 0)),
            pl.BlockSpec((BB, 3), lambda i: (i, 0)),
        ],
        out_shape=[
            jax.ShapeDtypeStruct((B, N), jnp.float32),
            jax.ShapeDtypeStruct((B, 3), jnp.float32),
        ],
    )(planes, d, Wc0, Wc1)


def kernel(x, d, gridWeight, Wc0, Wc1):
    idx = x.reshape(NW, N_DMA, DMA_B)
    gw8 = jnp.pad(gridWeight, ((0, 0), (0, 4)))
    planes = _sc_gather()(gw8, idx).reshape(7, B, N)
    sigma, color = _tc_dense(planes, d, Wc0, Wc1)
    return (sigma, color)


# speedup vs baseline: 3.5385x; 2.9465x over previous
import functools

import jax
import jax.numpy as jnp
from jax import lax
from jax.experimental import pallas as pl
from jax.experimental.pallas import tpu as pltpu
from jax.experimental.pallas import tpu_sc as plsc

B, N = 4096, 64
VOCAB = 1000000
TOTAL = B * N * 2
SAMPLES = B * N
NC, NS = 2, 16
NW = NC * NS
PER_W = TOTAL // NW
SAMP_W = SAMPLES // NW
DMA_B = 128
N_DMA = PER_W // DMA_B
FIRE = 8

CH_S = 2048
CH_R = CH_S * 2
N_CHUNK = PER_W // CH_R
DMA_PER_CHUNK = CH_R // DMA_B


def _sc_gather_body(table_hbm, idx_hbm, out_hbm, idx_v, rows_v, plane_v, sem):
    wid = lax.axis_index("s") * NC + lax.axis_index("c")
    pltpu.sync_copy(idx_hbm.at[wid], idx_v)
    lane = lax.iota(jnp.int32, 16)

    for chunk in range(N_CHUNK):
        dma0 = chunk * DMA_PER_CHUNK

        def gbody(g, carry):
            cps = [
                pltpu.async_copy(
                    table_hbm.at[idx_v.at[dma0 + g * FIRE + t]],
                    rows_v.at[pl.ds((g * FIRE + t) * DMA_B, DMA_B), :],
                    sem,
                )
                for t in range(FIRE)
            ]
            for cp in cps:
                cp.wait()
            return carry

        lax.fori_loop(0, DMA_PER_CHUNK // FIRE, gbody, 0)

        def rbody(j, carry):
            row0 = j * 32 + 2 * lane
            row1 = row0 + 1
            c0 = jnp.zeros((16,), jnp.int32)
            s1 = plsc.load_gather(rows_v, [row0, c0])
            s2 = plsc.load_gather(rows_v, [row1, c0])
            plane_v[0, pl.ds(j * 16, 16)] = s1 * s2
            for k in range(3):
                ck = jnp.full((16,), k + 1, jnp.int32)
                plane_v[1 + k, pl.ds(j * 16, 16)] = plsc.load_gather(
                    rows_v, [row0, ck]
                )
                plane_v[4 + k, pl.ds(j * 16, 16)] = plsc.load_gather(
                    rows_v, [row1, ck]
                )
            return carry

        lax.fori_loop(0, CH_S // 16, rbody, 0)

        base = wid * SAMP_W + chunk * CH_S
        cps = [
            pltpu.async_copy(plane_v.at[k], out_hbm.at[k, pl.ds(base, CH_S)], sem)
            for k in range(7)
        ]
        for cp in cps:
            cp.wait()


@functools.cache
def _sc_gather():
    return pl.kernel(
        _sc_gather_body,
        out_type=jax.ShapeDtypeStruct((7, SAMPLES), jnp.float32),
        mesh=plsc.VectorSubcoreMesh(
            core_axis_name="c", subcore_axis_name="s",
            num_cores=NC, num_subcores=NS,
        ),
        scratch_types=[
            pltpu.VMEM((N_DMA, DMA_B), jnp.int32),
            pltpu.VMEM((CH_R, 8), jnp.float32),
            pltpu.VMEM((7, CH_S), jnp.float32),
            pltpu.SemaphoreType.DMA,
        ],
        compiler_params=pltpu.CompilerParams(
            use_tc_tiling_on_sc=False, needs_layout_passes=False
        ),
    )


BB = 512


def _sigmoid(z):
    return 1.0 / (1.0 + jnp.exp(-z))


def _tc_body(planes_ref, d_ref, w0_ref, w1_ref, sigma_ref, color_ref):
    p = planes_ref[...]
    w0 = w0_ref[...]
    w1 = w1_ref[...]

    sigma = _sigmoid(p[0])
    sigma_ref[...] = sigma

    dproj = lax.dot_general(
        d_ref[...], w0[:, :16], (((1,), (1,)), ((), ())),
        preferred_element_type=jnp.float32,
    )

    hs = []
    for j in range(8):
        acc = dproj[:, j][:, None]
        for k in range(6):
            acc = acc + p[1 + k] * w0[j, 16 + k]
        hs.append(jnp.maximum(acc, 0.0))

    cs = []
    for i in range(3):
        accc = hs[0] * w1[i, 0]
        for j in range(1, 8):
            accc = accc + hs[j] * w1[i, j]
        cs.append(_sigmoid(accc))

    cp = 1.0 - sigma
    for sh in (1, 2, 4, 8, 16, 32):
        shifted = jnp.concatenate(
            [jnp.ones((BB, sh), jnp.float32), cp[:, :-sh]], axis=1
        )
        cp = cp * shifted
    tb = jnp.concatenate([jnp.ones((BB, 1), jnp.float32), cp[:, :-1]], axis=1)

    cw = tb * sigma
    color_ref[...] = jnp.stack(
        [jnp.sum(cw * cs[i], axis=1) for i in range(3)], axis=1
    )


def _tc_dense(planes, d, Wc0, Wc1):
    return pl.pallas_call(
        _tc_body,
        grid=(B // BB,),
        in_specs=[
            pl.BlockSpec((7, BB, N), lambda i: (0, i, 0)),
            pl.BlockSpec((BB, 16), lambda i: (i, 0)),
            pl.BlockSpec((8, 22), lambda i: (0, 0)),
            pl.BlockSpec((3, 8), lambda i: (0, 0)),
        ],
        out_specs=[
            pl.BlockSpec((BB, N), lambda i: (i, 0)),
            pl.BlockSpec((BB, 3), lambda i: (i, 0)),
        ],
        out_shape=[
            jax.ShapeDtypeStruct((B, N), jnp.float32),
            jax.ShapeDtypeStruct((B, 3), jnp.float32),
        ],
    )(planes, d, Wc0, Wc1)


def kernel(x, d, gridWeight, Wc0, Wc1):
    idx = x.reshape(NW, N_DMA, DMA_B)
    gw8 = jnp.concatenate([gridWeight, gridWeight], axis=1)
    planes = _sc_gather()(gw8, idx).reshape(7, B, N)
    sigma, color = _tc_dense(planes, d, Wc0, Wc1)
    return (sigma, color)
